# baseline (speedup 1.0000x reference)
import jax
import jax.numpy as jnp
from jax.experimental import pallas as pl
from jax.experimental.pallas import tpu as pltpu

_BM = 400
_NBUF = 3


def _mm_kernel(adj_hbm, x_ref, out_ref, bufs, xb_ref, sems):
    i = pl.program_id(0)
    nstep = pl.num_programs(0)
    bm = bufs.shape[1]

    @pl.when(i == 0)
    def _():
        xb_ref[...] = x_ref[...].astype(jnp.bfloat16)
        for b in range(_NBUF - 1):
            pltpu.make_async_copy(
                adj_hbm.at[pl.ds(b * bm, bm), :], bufs.at[b], sems.at[b]
            ).start()

    nxt = i + _NBUF - 1

    @pl.when(nxt < nstep)
    def _():
        slot = jax.lax.rem(nxt, _NBUF)
        pltpu.make_async_copy(
            adj_hbm.at[pl.ds(nxt * bm, bm), :], bufs.at[slot], sems.at[slot]
        ).start()

    slot = jax.lax.rem(i, _NBUF)
    pltpu.make_async_copy(
        adj_hbm.at[pl.ds(i * bm, bm), :], bufs.at[slot], sems.at[slot]
    ).wait()
    out_ref[...] = jnp.dot(bufs[slot].astype(jnp.bfloat16), xb_ref[...],
                           preferred_element_type=jnp.float32)


def kernel(adj, x):
    m, k = adj.shape
    _, n = x.shape
    bm = _BM if m % _BM == 0 else m
    return pl.pallas_call(
        _mm_kernel,
        grid=(m // bm,),
        in_specs=[
            pl.BlockSpec(memory_space=pl.ANY),
            pl.BlockSpec((k, n), lambda i: (0, 0)),
        ],
        out_specs=pl.BlockSpec((bm, n), lambda i: (i, 0)),
        out_shape=jax.ShapeDtypeStruct((m, n), jnp.float32),
        scratch_shapes=[
            pltpu.VMEM((_NBUF, bm, k), jnp.float32),
            pltpu.VMEM((k, n), jnp.bfloat16),
            pltpu.SemaphoreType.DMA((_NBUF,)),
        ],
        compiler_params=pltpu.CompilerParams(
            dimension_semantics=("arbitrary",),
        ),
    )(adj, x)


# speedup vs baseline: 1.0309x; 1.0309x over previous
import jax
import jax.numpy as jnp
from jax.experimental import pallas as pl
from jax.experimental.pallas import tpu as pltpu

_BK = 512


def _mm_kernel(adj_ref, x_ref, out_ref, xb_ref, *, nslab, tail):
    j = pl.program_id(0)
    bk = adj_ref.shape[1]

    @pl.when(j == 0)
    def _():
        xb_ref[...] = x_ref[...].astype(jnp.bfloat16)
        out_ref[...] = jnp.dot(adj_ref[...].astype(jnp.bfloat16),
                               xb_ref[pl.ds(0, bk), :],
                               preferred_element_type=jnp.float32)

    @pl.when((j > 0) & (j < nslab - 1))
    def _():
        out_ref[...] += jnp.dot(adj_ref[...].astype(jnp.bfloat16),
                                xb_ref[pl.ds(j * bk, bk), :],
                                preferred_element_type=jnp.float32)

    @pl.when(j == nslab - 1)
    def _():
        out_ref[...] += jnp.dot(adj_ref[:, :tail].astype(jnp.bfloat16),
                                xb_ref[pl.ds((nslab - 1) * bk, tail), :],
                                preferred_element_type=jnp.float32)


def kernel(adj, x):
    import functools
    m, k = adj.shape
    _, n = x.shape
    nslab = -(-k // _BK)
    tail = k - (nslab - 1) * _BK
    return pl.pallas_call(
        functools.partial(_mm_kernel, nslab=nslab, tail=tail),
        grid=(nslab,),
        in_specs=[
            pl.BlockSpec((m, _BK), lambda j: (0, j)),
            pl.BlockSpec((k, n), lambda j: (0, 0)),
        ],
        out_specs=pl.BlockSpec((m, n), lambda j: (0, 0),
                               pipeline_mode=pl.Buffered(buffer_count=1)),
        out_shape=jax.ShapeDtypeStruct((m, n), jnp.float32),
        scratch_shapes=[pltpu.VMEM((k, n), jnp.bfloat16)],
        compiler_params=pltpu.CompilerParams(
            dimension_semantics=("arbitrary",),
        ),
    )(adj, x)


# speedup vs baseline: 1.0402x; 1.0090x over previous
import jax
import jax.numpy as jnp
from jax.experimental import pallas as pl
from jax.experimental.pallas import tpu as pltpu

_BM = 400
_OGROUP = 5


def _mm_kernel(adj_ref, x_ref, out_ref, xb_ref):
    i = pl.program_id(0)

    @pl.when(i == 0)
    def _():
        xb_ref[...] = x_ref[...].astype(jnp.bfloat16)

    bm = adj_ref.shape[0]
    sub = jax.lax.rem(i, _OGROUP)
    out_ref[pl.ds(sub * bm, bm), :] = jnp.dot(
        adj_ref[...].astype(jnp.bfloat16), xb_ref[...],
        preferred_element_type=jnp.float32)


def kernel(adj, x):
    m, k = adj.shape
    _, n = x.shape
    bm = _BM if m % _BM == 0 else m
    return pl.pallas_call(
        _mm_kernel,
        grid=(m // bm,),
        in_specs=[
            pl.BlockSpec((bm, k), lambda i: (i, 0)),
            pl.BlockSpec((k, n), lambda i: (0, 0)),
        ],
        out_specs=pl.BlockSpec((bm * _OGROUP, n), lambda i: (i // _OGROUP, 0)),
        out_shape=jax.ShapeDtypeStruct((m, n), jnp.float32),
        scratch_shapes=[pltpu.VMEM((k, n), jnp.bfloat16)],
        compiler_params=pltpu.CompilerParams(
            dimension_semantics=("arbitrary",),
        ),
    )(adj, x)
